# baseline (speedup 1.0000x reference)
import jax
import jax.numpy as jnp
from jax.experimental import pallas as pl
from jax.experimental.pallas import tpu as pltpu

_LOG2E = 1.4426950408889634
_PREC = jax.lax.Precision.HIGHEST


def _dot(a, b):
    return jnp.dot(a, b, precision=_PREC)


def _dotg(a, b, dims):
    return jax.lax.dot_general(a, b, dims, precision=_PREC)


def _block_kernel(heads, hdim, r_blk, nx):
    hd = heads * hdim
    rn = r_blk * nx

    def body(xs_ref, pe_ref, lw1t_ref, ms1_ref, db1_ref, b1_ref,
             lw2t_ref, ms2_ref, db2_ref, b2_ref, fw_ref, out_ref):
        pe = pe_ref[0]
        lw1t = lw1t_ref[0]
        w0 = lw1t[0:1, :]
        wpe = lw1t[1:, :]
        ms1, db1, b1 = ms1_ref[0], db1_ref[0], b1_ref[0]
        ms2, db2, b2 = ms2_ref[0], db2_ref[0], b2_ref[0]
        lw2t, fw = lw2t_ref[0], fw_ref[0]
        dt = pe.dtype

        lane = jax.lax.broadcasted_iota(jnp.int32, (heads, hd), 1)
        row = jax.lax.broadcasted_iota(jnp.int32, (heads, hd), 0)
        headmask = (lane // hdim == row).astype(dt)
        plane = jax.lax.broadcasted_iota(jnp.int32, (r_blk, rn), 1)
        prow = jax.lax.broadcasted_iota(jnp.int32, (r_blk, rn), 0)
        poolbd = (plane // nx == prow).astype(dt)
        ones_col = jnp.ones((nx, 1), dtype=dt)

        def attn(h_all, ms, dbt, b):
            asrc_all = _dot(h_all, ms)
            adT = _dotg(
                dbt, h_all, (((1,), (1,)), ((), ())))
            outs = []
            for r in range(r_blk):
                s = slice(r * nx, (r + 1) * nx)
                ad = jnp.concatenate(
                    [adT[k:k + 1, s] for k in range(heads)], axis=1)
                logits = asrc_all[s, :] + ad
                e = jnp.maximum(logits, 0.2 * logits)
                ex = jnp.exp2(e)
                h_aug = jnp.concatenate([h_all[s, :], ones_col], axis=1)
                t = _dotg(
                    ex, h_aug, (((0,), (0,)), ((), ())))
                rden = 1.0 / (t[:, hd:hd + 1] + 1e-16)
                o = t[0:nx, 0:hd] * (headmask[0:1, :] * rden[0:nx])
                for k in range(1, heads):
                    o = o + t[k * nx:(k + 1) * nx, 0:hd] * (
                        headmask[k:k + 1, :] * rden[k * nx:(k + 1) * nx])
                outs.append(o + b)
            return jnp.concatenate(outs, axis=0)

        pe_h = _dot(pe, wpe)
        pe_h_all = jnp.concatenate([pe_h] * r_blk, axis=0)
        xcol = xs_ref[0]
        h1 = pe_h_all + xcol * w0
        x1 = attn(h1, ms1, db1, b1)
        h2 = _dot(x1, lw2t)
        x2 = attn(h2, ms2, db2, b2)
        pooled = _dot(poolbd, x2)
        y = _dotg(
            pooled, fw, (((1,), (1,)), ((), ())))
        out_ref[0] = y

    return body


def kernel(xs, pos_enc, lin_w1, src_w1, dst_w1, bias1,
           lin_w2, src_w2, dst_w2, bias2, final_w):
    bs, num_rows, num_xs = xs.shape
    enc = pos_enc.shape[-1]
    heads, hdim = src_w1.shape[2], src_w1.shape[3]
    hd = heads * hdim
    odim = final_w.shape[1]

    lw1_t = jnp.swapaxes(lin_w1, 1, 2)
    lw2_t = jnp.swapaxes(lin_w2, 1, 2)
    eye = jnp.eye(heads, dtype=xs.dtype)
    ms1 = jnp.einsum("bhd,hk->bhdk", src_w1.reshape(bs, heads, hdim) * _LOG2E, eye)
    ms1 = jnp.repeat(ms1.reshape(bs, hd, heads), num_xs, axis=2)
    ms2 = jnp.einsum("bhd,hk->bhdk", src_w2.reshape(bs, heads, hdim) * _LOG2E, eye)
    ms2 = jnp.repeat(ms2.reshape(bs, hd, heads), num_xs, axis=2)
    db1 = jnp.einsum("bhd,hk->bkhd", dst_w1.reshape(bs, heads, hdim) * _LOG2E,
                     eye).reshape(bs, heads, hd)
    db2 = jnp.einsum("bhd,hk->bkhd", dst_w2.reshape(bs, heads, hdim) * _LOG2E,
                     eye).reshape(bs, heads, hd)
    b1 = bias1[:, None, :]
    b2 = bias2[:, None, :]

    r_blk = 32
    grid = (bs, num_rows // r_blk)
    sample = lambda b, r: (b, 0, 0)
    xs_col = xs.reshape(bs, num_rows * num_xs, 1)
    out = pl.pallas_call(
        _block_kernel(heads, hdim, r_blk, num_xs),
        grid=grid,
        in_specs=[
            pl.BlockSpec((1, r_blk * num_xs, 1), lambda b, r: (b, r, 0)),
            pl.BlockSpec((1, num_xs, enc), sample),
            pl.BlockSpec((1, 1 + enc, hd), sample),
            pl.BlockSpec((1, hd, heads * num_xs), sample),
            pl.BlockSpec((1, heads, hd), sample),
            pl.BlockSpec((1, 1, hd), sample),
            pl.BlockSpec((1, hd, hd), sample),
            pl.BlockSpec((1, hd, heads * num_xs), sample),
            pl.BlockSpec((1, heads, hd), sample),
            pl.BlockSpec((1, 1, hd), sample),
            pl.BlockSpec((1, odim, hd), sample),
        ],
        out_specs=pl.BlockSpec((1, r_blk, odim), lambda b, r: (b, r, 0)),
        out_shape=jax.ShapeDtypeStruct((bs, num_rows, odim), xs.dtype),
        compiler_params=pltpu.CompilerParams(
            dimension_semantics=("parallel", "parallel")),
    )(xs_col, pos_enc, lw1_t, ms1, db1, b1, lw2_t, ms2, db2, b2, final_w)
    return out


# speedup vs baseline: 2.9196x; 2.9196x over previous
import jax
import jax.numpy as jnp
from jax.experimental import pallas as pl
from jax.experimental.pallas import tpu as pltpu

_LOG2E = 1.4426950408889634
_PREC = jax.lax.Precision.DEFAULT


def _dot(a, b):
    return jnp.dot(a, b, precision=_PREC)


def _dotg(a, b, dims):
    return jax.lax.dot_general(a, b, dims, precision=_PREC)


def _block_kernel(heads, hdim, r_blk, nx):
    hd = heads * hdim
    rn = r_blk * nx

    def body(xs_ref, pe_ref, lw1t_ref, ms1_ref, db1_ref, b1_ref,
             lw2t_ref, ms2_ref, db2_ref, b2_ref, fw_ref, out_ref):
        pe = pe_ref[0]
        lw1t = lw1t_ref[0]
        w0 = lw1t[0:1, :]
        wpe = lw1t[1:, :]
        ms1, db1, b1 = ms1_ref[0], db1_ref[0], b1_ref[0]
        ms2, db2, b2 = ms2_ref[0], db2_ref[0], b2_ref[0]
        lw2t, fw = lw2t_ref[0], fw_ref[0]
        dt = pe.dtype

        lane = jax.lax.broadcasted_iota(jnp.int32, (heads, hd), 1)
        row = jax.lax.broadcasted_iota(jnp.int32, (heads, hd), 0)
        headmask = (lane // hdim == row).astype(dt)
        plane = jax.lax.broadcasted_iota(jnp.int32, (r_blk, rn), 1)
        prow = jax.lax.broadcasted_iota(jnp.int32, (r_blk, rn), 0)
        poolbd = (plane // nx == prow).astype(dt)
        ones_col = jnp.ones((nx, 1), dtype=dt)

        def attn(h_all, ms, dbt, b):
            asrc_all = _dot(h_all, ms)
            adT = _dotg(
                dbt, h_all, (((1,), (1,)), ((), ())))
            outs = []
            for r in range(r_blk):
                s = slice(r * nx, (r + 1) * nx)
                ad = jnp.concatenate(
                    [adT[k:k + 1, s] for k in range(heads)], axis=1)
                logits = asrc_all[s, :] + ad
                e = jnp.maximum(logits, 0.2 * logits)
                ex = jnp.exp2(e)
                h_aug = jnp.concatenate([h_all[s, :], ones_col], axis=1)
                t = _dotg(
                    ex, h_aug, (((0,), (0,)), ((), ())))
                rden = 1.0 / (t[:, hd:hd + 1] + 1e-16)
                o = t[0:nx, 0:hd] * (headmask[0:1, :] * rden[0:nx])
                for k in range(1, heads):
                    o = o + t[k * nx:(k + 1) * nx, 0:hd] * (
                        headmask[k:k + 1, :] * rden[k * nx:(k + 1) * nx])
                outs.append(o + b)
            return jnp.concatenate(outs, axis=0)

        pe_h = _dot(pe, wpe)
        pe_h_all = jnp.concatenate([pe_h] * r_blk, axis=0)
        xcol = xs_ref[0]
        h1 = pe_h_all + xcol * w0
        x1 = attn(h1, ms1, db1, b1)
        h2 = _dot(x1, lw2t)
        x2 = attn(h2, ms2, db2, b2)
        pooled = _dot(poolbd, x2)
        y = _dotg(
            pooled, fw, (((1,), (1,)), ((), ())))
        out_ref[0] = y

    return body


def kernel(xs, pos_enc, lin_w1, src_w1, dst_w1, bias1,
           lin_w2, src_w2, dst_w2, bias2, final_w):
    bs, num_rows, num_xs = xs.shape
    enc = pos_enc.shape[-1]
    heads, hdim = src_w1.shape[2], src_w1.shape[3]
    hd = heads * hdim
    odim = final_w.shape[1]

    lw1_t = jnp.swapaxes(lin_w1, 1, 2)
    lw2_t = jnp.swapaxes(lin_w2, 1, 2)
    eye = jnp.eye(heads, dtype=xs.dtype)
    ms1 = jnp.einsum("bhd,hk->bhdk", src_w1.reshape(bs, heads, hdim) * _LOG2E, eye)
    ms1 = jnp.repeat(ms1.reshape(bs, hd, heads), num_xs, axis=2)
    ms2 = jnp.einsum("bhd,hk->bhdk", src_w2.reshape(bs, heads, hdim) * _LOG2E, eye)
    ms2 = jnp.repeat(ms2.reshape(bs, hd, heads), num_xs, axis=2)
    db1 = jnp.einsum("bhd,hk->bkhd", dst_w1.reshape(bs, heads, hdim) * _LOG2E,
                     eye).reshape(bs, heads, hd)
    db2 = jnp.einsum("bhd,hk->bkhd", dst_w2.reshape(bs, heads, hdim) * _LOG2E,
                     eye).reshape(bs, heads, hd)
    b1 = bias1[:, None, :]
    b2 = bias2[:, None, :]

    r_blk = 16
    grid = (bs, num_rows // r_blk)
    sample = lambda b, r: (b, 0, 0)
    xs_col = xs.reshape(bs, num_rows * num_xs, 1)
    out = pl.pallas_call(
        _block_kernel(heads, hdim, r_blk, num_xs),
        grid=grid,
        in_specs=[
            pl.BlockSpec((1, r_blk * num_xs, 1), lambda b, r: (b, r, 0)),
            pl.BlockSpec((1, num_xs, enc), sample),
            pl.BlockSpec((1, 1 + enc, hd), sample),
            pl.BlockSpec((1, hd, heads * num_xs), sample),
            pl.BlockSpec((1, heads, hd), sample),
            pl.BlockSpec((1, 1, hd), sample),
            pl.BlockSpec((1, hd, hd), sample),
            pl.BlockSpec((1, hd, heads * num_xs), sample),
            pl.BlockSpec((1, heads, hd), sample),
            pl.BlockSpec((1, 1, hd), sample),
            pl.BlockSpec((1, odim, hd), sample),
        ],
        out_specs=pl.BlockSpec((1, r_blk, odim), lambda b, r: (b, r, 0)),
        out_shape=jax.ShapeDtypeStruct((bs, num_rows, odim), xs.dtype),
        compiler_params=pltpu.CompilerParams(
            dimension_semantics=("parallel", "parallel")),
    )(xs_col, pos_enc, lw1_t, ms1, db1, b1, lw2_t, ms2, db2, b2, final_w)
    return out
